# Initial kernel scaffold; baseline (speedup 1.0000x reference)
import jax
import jax.numpy as jnp
from jax.experimental import pallas as pl


def kernel(feature, aggregated_vote_xyz, gu_W, gu_b, gu_g, gu_beta, rn1_W, rn1_b, rn2_W, rn2_b, bn1_g, bn1_b, rn3_W, rn3_b, bn2_g, bn2_b, rn4_W, rn4_b):
    raise NotImplementedError("write your pallas kernel here")


# speedup vs baseline: 8.1193x; 8.1193x over previous
import jax
import jax.numpy as jnp
from jax.experimental import pallas as pl

EPS = 1e-5
HI = jax.lax.Precision.HIGHEST


def _knn_kernel(cq_ref, ckT_ref, fq_ref, fall_ref,
                ru_ref, rubar_ref, idx_ref, mu_ref, S_ref):
    first = (pl.program_id(0) == 0) & (pl.program_id(1) == 0)

    @pl.when(first)
    def _():
        mu_ref[...] = jnp.zeros_like(mu_ref)
        S_ref[...] = jnp.zeros_like(S_ref)

    cq = cq_ref[0]
    ckT = ckT_ref[0]
    qt, n = cq.shape[0], ckT.shape[1]
    acc = None
    for k in range(3):
        diff = cq[:, k:k + 1] - ckT[k:k + 1, :]
        sq = diff * diff
        acc = sq if acc is None else acc + sq
    dist = -acc

    iota = jax.lax.broadcasted_iota(jnp.int32, (qt, n), 1)
    fq = fq_ref[0]
    fall = fall_ref[0]
    rus = []
    idx_cols = []
    for k in range(4):
        m = jnp.max(dist, axis=1, keepdims=True)
        cand = jnp.where(dist == m, iota, n)
        sel = jnp.min(cand, axis=1, keepdims=True)
        idx_cols.append(sel)
        if k > 0:
            onehot = (iota == sel).astype(jnp.float32)
            fsel = jnp.dot(onehot, fall, precision=HI,
                           preferred_element_type=jnp.float32)
            rus.append(fq + fsel)
        if k < 3:
            dist = jnp.where(iota == sel, -jnp.inf, dist)

    for p in range(3):
        ru_ref[0, p] = rus[p]
    rsum = rus[0] + rus[1] + rus[2]
    rubar_ref[0] = rsum * (1.0 / 3.0)
    idx_blk = jnp.concatenate(
        idx_cols + [jnp.zeros((qt, 4), jnp.int32)], axis=1)
    idx_ref[0] = idx_blk

    mu_ref[...] += jnp.sum(rsum, axis=0, keepdims=True)
    gram = None
    for p in range(3):
        g = jax.lax.dot_general(rus[p], rus[p], (((0,), (0,)), ((), ())),
                                precision=HI,
                                preferred_element_type=jnp.float32)
        gram = g if gram is None else gram + g
    S_ref[...] += gram


def _coeff1_kernel(musum_ref, S_ref, guWT_ref, gub_ref, gug_ref, gubeta_ref,
                   rn1WT_ref, rn1b_ref, rn2WT_ref, rn2b_ref, bn1g_ref,
                   bn1b_ref, minv_ref,
                   WeffT_ref, beff_ref, W2effT_ref, b2eff_ref):
    minv = minv_ref[0, 0]
    mu = musum_ref[...] * minv
    outer = jax.lax.dot_general(mu, mu, (((0,), (0,)), ((), ())),
                                precision=HI,
                                preferred_element_type=jnp.float32)
    cov = S_ref[...] * minv - outer

    guWT = guWT_ref[...]
    bg = jnp.dot(cov, guWT, precision=HI, preferred_element_type=jnp.float32)
    var_g = jnp.sum(guWT * bg, axis=0, keepdims=True)
    m_g = jnp.dot(mu, guWT, precision=HI,
                  preferred_element_type=jnp.float32) + gub_ref[...]
    a = gug_ref[...] / jnp.sqrt(var_g + EPS)
    d = gubeta_ref[...] - a * m_g
    WeffT_ref[...] = jnp.dot(guWT * a, rn1WT_ref[...], precision=HI,
                             preferred_element_type=jnp.float32)
    beff_ref[...] = jnp.dot(a * gub_ref[...] + d, rn1WT_ref[...],
                            precision=HI,
                            preferred_element_type=jnp.float32) + rn1b_ref[...]

    rn2WT = rn2WT_ref[...]
    br = jnp.dot(cov, rn2WT, precision=HI, preferred_element_type=jnp.float32)
    var_r = jnp.sum(rn2WT * br, axis=0, keepdims=True)
    m_r = jnp.dot(mu, rn2WT, precision=HI,
                  preferred_element_type=jnp.float32) + rn2b_ref[...]
    a1 = bn1g_ref[...] / jnp.sqrt(var_r + EPS)
    d1 = bn1b_ref[...] - a1 * m_r
    W2effT_ref[...] = rn2WT * a1
    b2eff_ref[...] = a1 * rn2b_ref[...] + d1


def _mlp1_kernel(x_ref, w_ref, b_ref, y_ref, muy_ref, Sy_ref):
    first = pl.program_id(0) == 0

    @pl.when(first)
    def _():
        muy_ref[...] = jnp.zeros_like(muy_ref)
        Sy_ref[...] = jnp.zeros_like(Sy_ref)

    y = jax.nn.relu(jnp.dot(x_ref[...], w_ref[...], precision=HI,
                            preferred_element_type=jnp.float32) + b_ref[...])
    y_ref[...] = y
    muy_ref[...] += jnp.sum(y, axis=0, keepdims=True)
    Sy_ref[...] += jax.lax.dot_general(y, y, (((0,), (0,)), ((), ())),
                                       precision=HI,
                                       preferred_element_type=jnp.float32)


def _coeff2_kernel(musum_ref, S_ref, rn3WT_ref, rn3b_ref, bn2g_ref, bn2b_ref,
                   minv_ref, W3effT_ref, b3eff_ref):
    minv = minv_ref[0, 0]
    mu = musum_ref[...] * minv
    outer = jax.lax.dot_general(mu, mu, (((0,), (0,)), ((), ())),
                                precision=HI,
                                preferred_element_type=jnp.float32)
    cov = S_ref[...] * minv - outer
    rn3WT = rn3WT_ref[...]
    br = jnp.dot(cov, rn3WT, precision=HI, preferred_element_type=jnp.float32)
    var_r = jnp.sum(rn3WT * br, axis=0, keepdims=True)
    m_r = jnp.dot(mu, rn3WT, precision=HI,
                  preferred_element_type=jnp.float32) + rn3b_ref[...]
    a2 = bn2g_ref[...] / jnp.sqrt(var_r + EPS)
    d2 = bn2b_ref[...] - a2 * m_r
    W3effT_ref[...] = rn3WT * a2
    b3eff_ref[...] = a2 * rn3b_ref[...] + d2


def _rnf_kernel(x_ref, w_ref, b_ref, out_ref):
    out_ref[...] = jnp.dot(x_ref[...], w_ref[...], precision=HI,
                           preferred_element_type=jnp.float32) + b_ref[...]


def _mlp2_kernel(y_ref, w3_ref, b3_ref, w4_ref, b4_ref, out_ref):
    r0 = jax.nn.relu(jnp.dot(y_ref[...], w3_ref[...], precision=HI,
                             preferred_element_type=jnp.float32) + b3_ref[...])
    l0 = jnp.sum(r0 * w4_ref[0:1, :], axis=1, keepdims=True) + b4_ref[:, 0:1]
    l1 = jnp.sum(r0 * w4_ref[1:2, :], axis=1, keepdims=True) + b4_ref[:, 1:2]
    out_ref[...] = jnp.concatenate([l0, l1], axis=1)


def kernel(feature, aggregated_vote_xyz, gu_W, gu_b, gu_g, gu_beta, rn1_W,
           rn1_b, rn2_W, rn2_b, bn1_g, bn1_b, rn3_W, rn3_b, bn2_g, bn2_b,
           rn4_W, rn4_b):
    bs, C, N = feature.shape
    P = 3
    QT = 256
    M = bs * N * P

    f32 = jnp.float32
    feat = jnp.transpose(feature, (0, 2, 1))
    xyz = aggregated_vote_xyz
    xyzT = jnp.transpose(xyz, (0, 2, 1))

    ru, rubar, idx8, musum, S = pl.pallas_call(
        _knn_kernel,
        grid=(bs, N // QT),
        in_specs=[
            pl.BlockSpec((1, QT, 3), lambda b, q: (b, q, 0)),
            pl.BlockSpec((1, 3, N), lambda b, q: (b, 0, 0)),
            pl.BlockSpec((1, QT, C), lambda b, q: (b, q, 0)),
            pl.BlockSpec((1, N, C), lambda b, q: (b, 0, 0)),
        ],
        out_specs=[
            pl.BlockSpec((1, P, QT, C), lambda b, q: (b, 0, q, 0)),
            pl.BlockSpec((1, QT, C), lambda b, q: (b, q, 0)),
            pl.BlockSpec((1, QT, 8), lambda b, q: (b, q, 0)),
            pl.BlockSpec((1, C), lambda b, q: (0, 0)),
            pl.BlockSpec((C, C), lambda b, q: (0, 0)),
        ],
        out_shape=[
            jax.ShapeDtypeStruct((bs, P, N, C), f32),
            jax.ShapeDtypeStruct((bs, N, C), f32),
            jax.ShapeDtypeStruct((bs, N, 8), jnp.int32),
            jax.ShapeDtypeStruct((1, C), f32),
            jax.ShapeDtypeStruct((C, C), f32),
        ],
    )(xyz, xyzT, feat, feat)

    idx_j = idx8[:, :, 1:4]

    minv = jnp.full((1, 1), 1.0 / M, f32)
    row = lambda v: v.reshape(1, -1)
    WeffT, beff, W2effT, b2eff = pl.pallas_call(
        _coeff1_kernel,
        out_shape=[
            jax.ShapeDtypeStruct((C, C), f32),
            jax.ShapeDtypeStruct((1, C), f32),
            jax.ShapeDtypeStruct((C, C), f32),
            jax.ShapeDtypeStruct((1, C), f32),
        ],
    )(musum, S, gu_W.T, row(gu_b), row(gu_g), row(gu_beta), rn1_W.T,
      row(rn1_b), rn2_W.T, row(rn2_b), row(bn1_g), row(bn1_b), minv)

    X = ru.reshape(bs * P * N, C)
    R = 2048
    y1, musum_y, Sy = pl.pallas_call(
        _mlp1_kernel,
        grid=(M // R,),
        in_specs=[
            pl.BlockSpec((R, C), lambda t: (t, 0)),
            pl.BlockSpec((C, C), lambda t: (0, 0)),
            pl.BlockSpec((1, C), lambda t: (0, 0)),
        ],
        out_specs=[
            pl.BlockSpec((R, C), lambda t: (t, 0)),
            pl.BlockSpec((1, C), lambda t: (0, 0)),
            pl.BlockSpec((C, C), lambda t: (0, 0)),
        ],
        out_shape=[
            jax.ShapeDtypeStruct((M, C), f32),
            jax.ShapeDtypeStruct((1, C), f32),
            jax.ShapeDtypeStruct((C, C), f32),
        ],
    )(X, W2effT, b2eff)

    W3effT, b3eff = pl.pallas_call(
        _coeff2_kernel,
        out_shape=[
            jax.ShapeDtypeStruct((C, C), f32),
            jax.ShapeDtypeStruct((1, C), f32),
        ],
    )(musum_y, Sy, rn3_W.T, row(rn3_b), row(bn2_g), row(bn2_b), minv)

    rub = rubar.reshape(bs * N, C)
    rnf = pl.pallas_call(
        _rnf_kernel,
        grid=(bs * N // R,),
        in_specs=[
            pl.BlockSpec((R, C), lambda t: (t, 0)),
            pl.BlockSpec((C, C), lambda t: (0, 0)),
            pl.BlockSpec((1, C), lambda t: (0, 0)),
        ],
        out_specs=pl.BlockSpec((R, C), lambda t: (t, 0)),
        out_shape=jax.ShapeDtypeStruct((bs * N, C), f32),
    )(rub, WeffT, beff)
    rn_feature = rnf.reshape(bs, N, C).transpose(0, 2, 1)

    logits = pl.pallas_call(
        _mlp2_kernel,
        grid=(M // R,),
        in_specs=[
            pl.BlockSpec((R, C), lambda t: (t, 0)),
            pl.BlockSpec((C, C), lambda t: (0, 0)),
            pl.BlockSpec((1, C), lambda t: (0, 0)),
            pl.BlockSpec((2, C), lambda t: (0, 0)),
            pl.BlockSpec((1, 2), lambda t: (0, 0)),
        ],
        out_specs=pl.BlockSpec((R, 2), lambda t: (t, 0)),
        out_shape=jax.ShapeDtypeStruct((M, 2), f32),
    )(y1, W3effT, b3eff, rn4_W, row(rn4_b))

    logits_0 = logits.reshape(bs, P, N, 2).transpose(0, 3, 2, 1).reshape(
        bs, 2, N * P)
    return (logits_0, rn_feature, idx_j)


# speedup vs baseline: 11.1381x; 1.3718x over previous
import jax
import jax.numpy as jnp
from jax.experimental import pallas as pl

EPS = 1e-5
HI = jax.lax.Precision.HIGHEST


def _knn_kernel(cq_ref, ckT_ref, fq_ref, fhi_ref, flo_ref,
                ru_ref, rubar_ref, idx_ref, mu_ref, S_ref):
    first = (pl.program_id(0) == 0) & (pl.program_id(1) == 0)

    @pl.when(first)
    def _():
        mu_ref[...] = jnp.zeros_like(mu_ref)
        S_ref[...] = jnp.zeros_like(S_ref)

    cq = cq_ref[0]
    ckT = ckT_ref[0]
    qt, n = cq.shape[0], ckT.shape[1]
    acc = None
    for k in range(3):
        diff = cq[:, k:k + 1] - ckT[k:k + 1, :]
        sq = diff * diff
        acc = sq if acc is None else acc + sq
    dist = -acc

    iota = jax.lax.broadcasted_iota(jnp.int32, (qt, n), 1)
    fq = fq_ref[0]
    fhi = fhi_ref[0]
    flo = flo_ref[0]
    rus = []
    idx_cols = []
    for k in range(4):
        if k == 0:
            m = jnp.zeros((qt, 1), jnp.float32)
        else:
            m = jnp.max(dist, axis=1, keepdims=True)
        cand = jnp.where(dist == m, iota, n)
        sel = jnp.min(cand, axis=1, keepdims=True)
        idx_cols.append(sel)
        if k > 0:
            onehot = (iota == sel).astype(jnp.bfloat16)
            fsel = (jnp.dot(onehot, fhi, preferred_element_type=jnp.float32)
                    + jnp.dot(onehot, flo,
                              preferred_element_type=jnp.float32))
            rus.append(fq + fsel)
        if k < 3:
            dist = jnp.where(iota == sel, -jnp.inf, dist)

    for p in range(3):
        ru_ref[0, p] = rus[p]
    rsum = rus[0] + rus[1] + rus[2]
    rubar_ref[0] = rsum * (1.0 / 3.0)
    idx_blk = jnp.concatenate(
        idx_cols + [jnp.zeros((qt, 4), jnp.int32)], axis=1)
    idx_ref[0] = idx_blk

    mu_ref[...] += jnp.sum(rsum, axis=0, keepdims=True)
    gram = None
    for p in range(3):
        g = jax.lax.dot_general(rus[p], rus[p], (((0,), (0,)), ((), ())),
                                precision=HI,
                                preferred_element_type=jnp.float32)
        gram = g if gram is None else gram + g
    S_ref[...] += gram


def _coeff1_kernel(musum_ref, S_ref, guWT_ref, gub_ref, gug_ref, gubeta_ref,
                   rn1WT_ref, rn1b_ref, rn2WT_ref, rn2b_ref, bn1g_ref,
                   bn1b_ref, minv_ref,
                   WeffT_ref, beff_ref, W2effT_ref, b2eff_ref):
    minv = minv_ref[0, 0]
    mu = musum_ref[...] * minv
    outer = jax.lax.dot_general(mu, mu, (((0,), (0,)), ((), ())),
                                precision=HI,
                                preferred_element_type=jnp.float32)
    cov = S_ref[...] * minv - outer

    guWT = guWT_ref[...]
    bg = jnp.dot(cov, guWT, precision=HI, preferred_element_type=jnp.float32)
    var_g = jnp.sum(guWT * bg, axis=0, keepdims=True)
    m_g = jnp.dot(mu, guWT, precision=HI,
                  preferred_element_type=jnp.float32) + gub_ref[...]
    a = gug_ref[...] / jnp.sqrt(var_g + EPS)
    d = gubeta_ref[...] - a * m_g
    WeffT_ref[...] = jnp.dot(guWT * a, rn1WT_ref[...], precision=HI,
                             preferred_element_type=jnp.float32)
    beff_ref[...] = jnp.dot(a * gub_ref[...] + d, rn1WT_ref[...],
                            precision=HI,
                            preferred_element_type=jnp.float32) + rn1b_ref[...]

    rn2WT = rn2WT_ref[...]
    br = jnp.dot(cov, rn2WT, precision=HI, preferred_element_type=jnp.float32)
    var_r = jnp.sum(rn2WT * br, axis=0, keepdims=True)
    m_r = jnp.dot(mu, rn2WT, precision=HI,
                  preferred_element_type=jnp.float32) + rn2b_ref[...]
    a1 = bn1g_ref[...] / jnp.sqrt(var_r + EPS)
    d1 = bn1b_ref[...] - a1 * m_r
    W2effT_ref[...] = rn2WT * a1
    b2eff_ref[...] = a1 * rn2b_ref[...] + d1


def _mlp1_kernel(x_ref, w_ref, b_ref, y_ref, muy_ref, Sy_ref):
    first = pl.program_id(0) == 0

    @pl.when(first)
    def _():
        muy_ref[...] = jnp.zeros_like(muy_ref)
        Sy_ref[...] = jnp.zeros_like(Sy_ref)

    y = jax.nn.relu(jnp.dot(x_ref[...], w_ref[...], precision=HI,
                            preferred_element_type=jnp.float32) + b_ref[...])
    y_ref[...] = y
    muy_ref[...] += jnp.sum(y, axis=0, keepdims=True)
    Sy_ref[...] += jax.lax.dot_general(y, y, (((0,), (0,)), ((), ())),
                                       precision=HI,
                                       preferred_element_type=jnp.float32)


def _coeff2_kernel(musum_ref, S_ref, rn3WT_ref, rn3b_ref, bn2g_ref, bn2b_ref,
                   minv_ref, W3effT_ref, b3eff_ref):
    minv = minv_ref[0, 0]
    mu = musum_ref[...] * minv
    outer = jax.lax.dot_general(mu, mu, (((0,), (0,)), ((), ())),
                                precision=HI,
                                preferred_element_type=jnp.float32)
    cov = S_ref[...] * minv - outer
    rn3WT = rn3WT_ref[...]
    br = jnp.dot(cov, rn3WT, precision=HI, preferred_element_type=jnp.float32)
    var_r = jnp.sum(rn3WT * br, axis=0, keepdims=True)
    m_r = jnp.dot(mu, rn3WT, precision=HI,
                  preferred_element_type=jnp.float32) + rn3b_ref[...]
    a2 = bn2g_ref[...] / jnp.sqrt(var_r + EPS)
    d2 = bn2b_ref[...] - a2 * m_r
    W3effT_ref[...] = rn3WT * a2
    b3eff_ref[...] = a2 * rn3b_ref[...] + d2


def _rnf_kernel(x_ref, w_ref, b_ref, out_ref):
    out_ref[...] = jnp.dot(x_ref[...], w_ref[...], precision=HI,
                           preferred_element_type=jnp.float32) + b_ref[...]


def _mlp2_kernel(y_ref, w3_ref, b3_ref, w4_ref, b4_ref, out_ref):
    r0 = jax.nn.relu(jnp.dot(y_ref[...], w3_ref[...], precision=HI,
                             preferred_element_type=jnp.float32) + b3_ref[...])
    l0 = jnp.sum(r0 * w4_ref[0:1, :], axis=1, keepdims=True) + b4_ref[:, 0:1]
    l1 = jnp.sum(r0 * w4_ref[1:2, :], axis=1, keepdims=True) + b4_ref[:, 1:2]
    out_ref[...] = jnp.concatenate([l0, l1], axis=1)


def kernel(feature, aggregated_vote_xyz, gu_W, gu_b, gu_g, gu_beta, rn1_W,
           rn1_b, rn2_W, rn2_b, bn1_g, bn1_b, rn3_W, rn3_b, bn2_g, bn2_b,
           rn4_W, rn4_b):
    bs, C, N = feature.shape
    P = 3
    QT = 256
    M = bs * N * P

    f32 = jnp.float32
    feat = jnp.transpose(feature, (0, 2, 1))
    fhi = feat.astype(jnp.bfloat16)
    flo = (feat - fhi.astype(f32)).astype(jnp.bfloat16)
    xyz = aggregated_vote_xyz
    xyzT = jnp.transpose(xyz, (0, 2, 1))

    ru, rubar, idx8, musum, S = pl.pallas_call(
        _knn_kernel,
        grid=(bs, N // QT),
        in_specs=[
            pl.BlockSpec((1, QT, 3), lambda b, q: (b, q, 0)),
            pl.BlockSpec((1, 3, N), lambda b, q: (b, 0, 0)),
            pl.BlockSpec((1, QT, C), lambda b, q: (b, q, 0)),
            pl.BlockSpec((1, N, C), lambda b, q: (b, 0, 0)),
            pl.BlockSpec((1, N, C), lambda b, q: (b, 0, 0)),
        ],
        out_specs=[
            pl.BlockSpec((1, P, QT, C), lambda b, q: (b, 0, q, 0)),
            pl.BlockSpec((1, QT, C), lambda b, q: (b, q, 0)),
            pl.BlockSpec((1, QT, 8), lambda b, q: (b, q, 0)),
            pl.BlockSpec((1, C), lambda b, q: (0, 0)),
            pl.BlockSpec((C, C), lambda b, q: (0, 0)),
        ],
        out_shape=[
            jax.ShapeDtypeStruct((bs, P, N, C), f32),
            jax.ShapeDtypeStruct((bs, N, C), f32),
            jax.ShapeDtypeStruct((bs, N, 8), jnp.int32),
            jax.ShapeDtypeStruct((1, C), f32),
            jax.ShapeDtypeStruct((C, C), f32),
        ],
    )(xyz, xyzT, feat, fhi, flo)

    idx_j = idx8[:, :, 1:4]

    minv = jnp.full((1, 1), 1.0 / M, f32)
    row = lambda v: v.reshape(1, -1)
    WeffT, beff, W2effT, b2eff = pl.pallas_call(
        _coeff1_kernel,
        out_shape=[
            jax.ShapeDtypeStruct((C, C), f32),
            jax.ShapeDtypeStruct((1, C), f32),
            jax.ShapeDtypeStruct((C, C), f32),
            jax.ShapeDtypeStruct((1, C), f32),
        ],
    )(musum, S, gu_W.T, row(gu_b), row(gu_g), row(gu_beta), rn1_W.T,
      row(rn1_b), rn2_W.T, row(rn2_b), row(bn1_g), row(bn1_b), minv)

    X = ru.reshape(bs * P * N, C)
    R = 2048
    y1, musum_y, Sy = pl.pallas_call(
        _mlp1_kernel,
        grid=(M // R,),
        in_specs=[
            pl.BlockSpec((R, C), lambda t: (t, 0)),
            pl.BlockSpec((C, C), lambda t: (0, 0)),
            pl.BlockSpec((1, C), lambda t: (0, 0)),
        ],
        out_specs=[
            pl.BlockSpec((R, C), lambda t: (t, 0)),
            pl.BlockSpec((1, C), lambda t: (0, 0)),
            pl.BlockSpec((C, C), lambda t: (0, 0)),
        ],
        out_shape=[
            jax.ShapeDtypeStruct((M, C), f32),
            jax.ShapeDtypeStruct((1, C), f32),
            jax.ShapeDtypeStruct((C, C), f32),
        ],
    )(X, W2effT, b2eff)

    W3effT, b3eff = pl.pallas_call(
        _coeff2_kernel,
        out_shape=[
            jax.ShapeDtypeStruct((C, C), f32),
            jax.ShapeDtypeStruct((1, C), f32),
        ],
    )(musum_y, Sy, rn3_W.T, row(rn3_b), row(bn2_g), row(bn2_b), minv)

    rub = rubar.reshape(bs * N, C)
    rnf = pl.pallas_call(
        _rnf_kernel,
        grid=(bs * N // R,),
        in_specs=[
            pl.BlockSpec((R, C), lambda t: (t, 0)),
            pl.BlockSpec((C, C), lambda t: (0, 0)),
            pl.BlockSpec((1, C), lambda t: (0, 0)),
        ],
        out_specs=pl.BlockSpec((R, C), lambda t: (t, 0)),
        out_shape=jax.ShapeDtypeStruct((bs * N, C), f32),
    )(rub, WeffT, beff)
    rn_feature = rnf.reshape(bs, N, C).transpose(0, 2, 1)

    logits = pl.pallas_call(
        _mlp2_kernel,
        grid=(M // R,),
        in_specs=[
            pl.BlockSpec((R, C), lambda t: (t, 0)),
            pl.BlockSpec((C, C), lambda t: (0, 0)),
            pl.BlockSpec((1, C), lambda t: (0, 0)),
            pl.BlockSpec((2, C), lambda t: (0, 0)),
            pl.BlockSpec((1, 2), lambda t: (0, 0)),
        ],
        out_specs=pl.BlockSpec((R, 2), lambda t: (t, 0)),
        out_shape=jax.ShapeDtypeStruct((M, 2), f32),
    )(y1, W3effT, b3eff, rn4_W, row(rn4_b))

    logits_0 = logits.reshape(bs, P, N, 2).transpose(0, 3, 2, 1).reshape(
        bs, 2, N * P)
    return (logits_0, rn_feature, idx_j)
